# Initial kernel scaffold; baseline (speedup 1.0000x reference)
import jax
import jax.numpy as jnp
from jax.experimental import pallas as pl


def kernel(data, ids, space_pts, time_pts, query_pts, eig, params):
    raise NotImplementedError("write your pallas kernel here")


# speedup vs baseline: 2.7409x; 2.7409x over previous
import functools

import numpy as np
import jax
import jax.numpy as jnp
from jax import lax
from jax.experimental import pallas as pl
from jax.experimental.pallas import tpu as pltpu
from jax.experimental.pallas import tpu_sc as plsc

_EIG = 16
_K = 16
_NW = 32
_CH = 128
_GRP = 4
_D = 128

_F2PI = (np.float64(np.float32(2.0 * np.pi)) *
         (2.0 ** np.arange(16))).astype(np.float32)


def _full_spec(shape):
    nd = len(shape)
    return pl.BlockSpec(shape, lambda *_: (0,) * nd)


def _sc_gather(table, idx, d, n_chunks, group):
    m = idx.shape[0]
    per_w = n_chunks * _CH
    assert m == _NW * per_w, (m, n_chunks)
    assert n_chunks % group == 0
    n_groups = n_chunks // group
    mesh = plsc.VectorSubcoreMesh(core_axis_name="c", subcore_axis_name="s")

    @functools.partial(
        pl.kernel,
        mesh=mesh,
        out_type=jax.ShapeDtypeStruct((m, d), jnp.float32),
        scratch_types=[
            pltpu.VMEM((group, _CH), jnp.int32),
            pltpu.VMEM((group, _CH, d), jnp.float32),
            pltpu.SemaphoreType.DMA,
        ],
    )
    def gather_kernel(table_hbm, idx_hbm, out_hbm, idx_v, rows_v, sem):
        wid = lax.axis_index("s") * 2 + lax.axis_index("c")
        base = wid * per_w

        def body(g, carry):
            goff = base + g * (group * _CH)
            handles = []
            for b in range(group):
                pltpu.sync_copy(idx_hbm.at[pl.ds(goff + b * _CH, _CH)],
                                idx_v.at[b])
                handles.append(
                    pltpu.async_copy(table_hbm.at[idx_v.at[b]],
                                     rows_v.at[b], sem))
            for h in handles:
                h.wait()
            for b in range(group):
                pltpu.sync_copy(rows_v.at[b],
                                out_hbm.at[pl.ds(goff + b * _CH, _CH)])
            return carry

        lax.fori_loop(0, n_groups, body, 0)

    return gather_kernel(table, idx)


def _spectral_coeffs(x, e, theta2):
    bsz, n, c = x.shape

    def body(x_ref, e_ref, th_ref, out_ref):
        co = lax.dot_general(e_ref[0], x_ref[0], (((0,), (0,)), ((), ())),
                             preferred_element_type=jnp.float32)
        out_ref[0] = co * th_ref[...]

    return pl.pallas_call(
        body,
        grid=(bsz,),
        in_specs=[
            pl.BlockSpec((1, n, c), lambda i: (i, 0, 0)),
            pl.BlockSpec((1, n, _EIG), lambda i: (i, 0, 0)),
            _full_spec((_EIG, 1)),
        ],
        out_specs=pl.BlockSpec((1, _EIG, c), lambda i: (i, 0, 0)),
        out_shape=jax.ShapeDtypeStruct((bsz, _EIG, c), jnp.float32),
    )(x, e, theta2)


def _layer_pre(x, filt, e, sp, w, blk):
    bsz, n, c = x.shape

    def body(x_ref, f_ref, e_ref, sp_ref, w1x, w1p, w1s, b1, w2, b2,
             px, pn, pb, nei_ref, fp_ref):
        xs = x_ref[0]
        pt = jnp.dot(e_ref[0], f_ref[0], preferred_element_type=jnp.float32)
        h = (jnp.dot(xs, w1x[...], preferred_element_type=jnp.float32)
             + jnp.dot(pt, w1p[...], preferred_element_type=jnp.float32)
             + jnp.dot(sp_ref[0], w1s[...], preferred_element_type=jnp.float32)
             + b1[...])
        h = jnp.maximum(h, 0.0)
        nei = jnp.dot(h, w2[...], preferred_element_type=jnp.float32) + b2[...]
        nei_ref[0] = nei
        fp_ref[0] = (jnp.dot(xs, px[...], preferred_element_type=jnp.float32)
                     + jnp.dot(nei, pn[...], preferred_element_type=jnp.float32)
                     + pb[...])

    nb = n // blk
    return pl.pallas_call(
        body,
        grid=(bsz, nb),
        in_specs=[
            pl.BlockSpec((1, blk, c), lambda i, j: (i, j, 0)),
            pl.BlockSpec((1, _EIG, c), lambda i, j: (i, 0, 0)),
            pl.BlockSpec((1, blk, _EIG), lambda i, j: (i, j, 0)),
            pl.BlockSpec((1, blk, 3), lambda i, j: (i, j, 0)),
            _full_spec(w['w1x'].shape), _full_spec(w['w1p'].shape),
            _full_spec(w['w1s'].shape), _full_spec(w['b1'].shape),
            _full_spec(w['w2'].shape), _full_spec(w['b2'].shape),
            _full_spec(w['px'].shape), _full_spec(w['pn'].shape),
            _full_spec(w['pb'].shape),
        ],
        out_specs=[
            pl.BlockSpec((1, blk, 64), lambda i, j: (i, j, 0)),
            pl.BlockSpec((1, blk, _EIG), lambda i, j: (i, j, 0)),
        ],
        out_shape=[
            jax.ShapeDtypeStruct((bsz, n, 64), jnp.float32),
            jax.ShapeDtypeStruct((bsz, n, _EIG), jnp.float32),
        ],
    )(x, filt, e, sp, w['w1x'], w['w1p'], w['w1s'], w['b1'], w['w2'], w['b2'],
      w['px'], w['pn'], w['pb'])


def _layer_post(x, nei, gath, t, w, blk):
    bsz, n, c = x.shape

    def body(x_ref, nei_ref, g_ref, t_ref, fq_ref, ws, wc, wb1, ww2, wb2,
             fw1, fb1, fw2, fb2, c1x, c1n, c1t, cb1, c2w, cb2, out_ref):
        g = g_ref[0]
        fn = g[:, :, :_EIG]
        tn = g[:, :, _EIG:_EIG + 1]
        dt = tn - t_ref[0][:, :, None]
        ang = dt * fq_ref[...][None]
        s = jnp.sin(ang).reshape(blk * _K, 16)
        co = jnp.cos(ang).reshape(blk * _K, 16)
        h = (jnp.dot(s, ws[...], preferred_element_type=jnp.float32)
             + jnp.dot(co, wc[...], preferred_element_type=jnp.float32)
             + wb1[...])
        h = jnp.maximum(h, 0.0)
        wgt = (jnp.dot(h, ww2[...], preferred_element_type=jnp.float32)
               + wb2[...]).reshape(blk, _K, _EIG)
        agg = jnp.sum(wgt * fn, axis=1)
        h2 = jnp.maximum(
            jnp.dot(agg, fw1[...], preferred_element_type=jnp.float32)
            + fb1[...], 0.0)
        tnei = (jnp.dot(h2, fw2[...], preferred_element_type=jnp.float32)
                + fb2[...])
        hh = (jnp.dot(x_ref[0], c1x[...], preferred_element_type=jnp.float32)
              + jnp.dot(nei_ref[0], c1n[...], preferred_element_type=jnp.float32)
              + jnp.dot(tnei, c1t[...], preferred_element_type=jnp.float32)
              + cb1[...])
        hh = jnp.maximum(hh, 0.0)
        out_ref[0] = (jnp.dot(hh, c2w[...], preferred_element_type=jnp.float32)
                      + cb2[...])

    nb = n // blk
    return pl.pallas_call(
        body,
        grid=(bsz, nb),
        in_specs=[
            pl.BlockSpec((1, blk, c), lambda i, j: (i, j, 0)),
            pl.BlockSpec((1, blk, 64), lambda i, j: (i, j, 0)),
            pl.BlockSpec((1, blk, _K, _D), lambda i, j: (i, j, 0, 0)),
            pl.BlockSpec((1, blk, 1), lambda i, j: (i, j, 0)),
            _full_spec((1, 16)),
            _full_spec(w['ws'].shape), _full_spec(w['wc'].shape),
            _full_spec(w['wb1'].shape), _full_spec(w['ww2'].shape),
            _full_spec(w['wb2'].shape), _full_spec(w['fw1'].shape),
            _full_spec(w['fb1'].shape), _full_spec(w['fw2'].shape),
            _full_spec(w['fb2'].shape), _full_spec(w['c1x'].shape),
            _full_spec(w['c1n'].shape), _full_spec(w['c1t'].shape),
            _full_spec(w['cb1'].shape), _full_spec(w['c2'].shape),
            _full_spec(w['cb2'].shape),
        ],
        out_specs=pl.BlockSpec((1, blk, 128), lambda i, j: (i, j, 0)),
        out_shape=jax.ShapeDtypeStruct((bsz, n, 128), jnp.float32),
    )(x, nei, gath, t, jnp.asarray(_F2PI).reshape(1, 16),
      w['ws'], w['wc'], w['wb1'], w['ww2'], w['wb2'],
      w['fw1'], w['fb1'], w['fw2'], w['fb2'], w['c1x'], w['c1n'], w['c1t'],
      w['cb1'], w['c2'], w['cb2'])


def _target_proj(x, pw, pb, blk):
    bsz, n, c = x.shape
    d = pw.shape[1]

    def body(x_ref, pw_ref, pb_ref, out_ref):
        out_ref[0] = (jnp.dot(x_ref[0], pw_ref[...],
                              preferred_element_type=jnp.float32)
                      + pb_ref[...])

    return pl.pallas_call(
        body,
        grid=(bsz, n // blk),
        in_specs=[
            pl.BlockSpec((1, blk, c), lambda i, j: (i, j, 0)),
            _full_spec(pw.shape), _full_spec(pb.shape),
        ],
        out_specs=pl.BlockSpec((1, blk, d), lambda i, j: (i, j, 0)),
        out_shape=jax.ShapeDtypeStruct((bsz, n, d), jnp.float32),
    )(x, pw, pb)


def _target_conv(gath, qt, w, blk):
    bsz, nq = qt.shape[0], qt.shape[1]

    def body(g_ref, t_ref, fq_ref, ws, wc, wb1, ww2, wb2, fw1, fb1, fw2, fb2,
             out_ref):
        g = g_ref[0]
        fn = g[:, :, :32]
        tn = g[:, :, 32:33]
        dt = tn - t_ref[0][:, :, None]
        ang = dt * fq_ref[...][None]
        s = jnp.sin(ang).reshape(blk * _K, 16)
        co = jnp.cos(ang).reshape(blk * _K, 16)
        h = (jnp.dot(s, ws[...], preferred_element_type=jnp.float32)
             + jnp.dot(co, wc[...], preferred_element_type=jnp.float32)
             + wb1[...])
        h = jnp.maximum(h, 0.0)
        wgt = (jnp.dot(h, ww2[...], preferred_element_type=jnp.float32)
               + wb2[...]).reshape(blk, _K, 32)
        agg = jnp.sum(wgt * fn, axis=1)
        h2 = jnp.maximum(
            jnp.dot(agg, fw1[...], preferred_element_type=jnp.float32)
            + fb1[...], 0.0)
        out_ref[0] = (jnp.dot(h2, fw2[...], preferred_element_type=jnp.float32)
                      + fb2[...])

    return pl.pallas_call(
        body,
        grid=(bsz, nq // blk),
        in_specs=[
            pl.BlockSpec((1, blk, _K, _D), lambda i, j: (i, j, 0, 0)),
            pl.BlockSpec((1, blk, 1), lambda i, j: (i, j, 0)),
            _full_spec((1, 16)),
            _full_spec(w['ws'].shape), _full_spec(w['wc'].shape),
            _full_spec(w['wb1'].shape), _full_spec(w['ww2'].shape),
            _full_spec(w['wb2'].shape), _full_spec(w['fw1'].shape),
            _full_spec(w['fb1'].shape), _full_spec(w['fw2'].shape),
            _full_spec(w['fb2'].shape),
        ],
        out_specs=pl.BlockSpec((1, blk, 128), lambda i, j: (i, j, 0)),
        out_shape=jax.ShapeDtypeStruct((bsz, nq, 128), jnp.float32),
    )(gath, qt, jnp.asarray(_F2PI).reshape(1, 16),
      w['ws'], w['wc'], w['wb1'], w['ww2'], w['wb2'],
      w['fw1'], w['fb1'], w['fw2'], w['fb2'])


def _row(b):
    return b.reshape(1, -1)


def _prep_layer(lp, c):
    s, t, cb = lp['spec'], lp['time'], lp['comb']
    w = {}
    w['w1x'] = s['Ws'][0][:c]
    w['w1p'] = s['Ws'][0][c:2 * c]
    w['w1s'] = s['Ws'][0][2 * c:]
    w['b1'] = _row(s['bs'][0])
    w['w2'] = s['Ws'][1]
    w['b2'] = _row(s['bs'][1])
    w['px'] = t['proj_W'][:c]
    w['pn'] = t['proj_W'][c:]
    w['pb'] = _row(t['proj_b'])
    w['ws'] = t['w_Ws'][0][:16]
    w['wc'] = t['w_Ws'][0][16:]
    w['wb1'] = _row(t['w_bs'][0])
    w['ww2'] = t['w_Ws'][1]
    w['wb2'] = _row(t['w_bs'][1])
    w['fw1'] = t['f_Ws'][0]
    w['fb1'] = _row(t['f_bs'][0])
    w['fw2'] = t['f_Ws'][1]
    w['fb2'] = _row(t['f_bs'][1])
    w['c1x'] = cb['Ws'][0][:c]
    w['c1n'] = cb['Ws'][0][c:c + 64]
    w['c1t'] = cb['Ws'][0][c + 64:]
    w['cb1'] = _row(cb['bs'][0])
    w['c2'] = cb['Ws'][1]
    w['cb2'] = _row(cb['bs'][1])
    return w


def _pad_idx(idx, n_chunks):
    m = _NW * n_chunks * _CH
    return jnp.concatenate(
        [idx, jnp.zeros((m - idx.shape[0],), jnp.int32)])


def kernel(data, ids, space_pts, time_pts, query_pts, eig, params):
    bsz, n, _ = data.shape
    nq = query_pts.shape[1]

    k_t = time_pts[..., 0]
    order = jnp.argsort(k_t, axis=1)
    sorted_t = jnp.take_along_axis(k_t, order, axis=1)

    def neighbor_idx(q_t):
        pos = jax.vmap(lambda s, q: jnp.searchsorted(s, q))(sorted_t, q_t)
        start = jnp.clip(pos - _K // 2, 0, n - _K)
        win = start[:, :, None] + jnp.arange(_K)[None, None, :]
        return jax.vmap(lambda o, wn: o[wn])(order, win)

    boff = (jnp.arange(bsz, dtype=jnp.int32) * n)[:, None, None]
    idx_self = (neighbor_idx(k_t).astype(jnp.int32) + boff).reshape(-1)
    idx_tgt = (neighbor_idx(query_pts[..., 0]).astype(jnp.int32)
               + boff).reshape(-1)
    nc_self = (idx_self.shape[0] + _NW * _CH - 1) // (_NW * _CH)
    nc_self += (-nc_self) % _GRP
    nc_tgt = (idx_tgt.shape[0] + _NW * _CH - 1) // (_NW * _CH)
    nc_tgt += (-nc_tgt) % _GRP
    idx_self = _pad_idx(idx_self, nc_self)
    idx_tgt = _pad_idx(idx_tgt, nc_tgt)

    nc_e = (bsz * n + _NW * _CH - 1) // (_NW * _CH)
    nc_e += (-nc_e) % _GRP
    idx_e = _pad_idx(ids.reshape(-1), nc_e)

    eig_pad = jnp.concatenate(
        [eig, jnp.zeros((eig.shape[0], _D - _EIG), jnp.float32)], axis=-1)
    e = _sc_gather(eig_pad, idx_e, _D, nc_e, _GRP)[:bsz * n, :_EIG]
    e = e.reshape(bsz, n, _EIG)

    blk = 1000
    blk_post = 400
    zpad_self = jnp.zeros((bsz, n, _D - _EIG - 1), jnp.float32)
    zpad_tgt = jnp.zeros((bsz, n, _D - 33), jnp.float32)
    x = data
    for lp in params['layers']:
        c = x.shape[2]
        w = _prep_layer(lp, c)
        filt = _spectral_coeffs(x, e, lp['spec']['theta'].reshape(_EIG, 1))
        nei, fproj = _layer_pre(x, filt, e, space_pts, w, blk)
        table = jnp.concatenate([fproj, time_pts, zpad_self], axis=-1)
        gath = _sc_gather(table.reshape(bsz * n, _D), idx_self, _D,
                          nc_self, _GRP)
        gath = gath[:bsz * n * _K].reshape(bsz, n, _K, _D)
        x = _layer_post(x, nei, gath, time_pts, w, blk_post)

    tg = params['target']
    fproj = _target_proj(x, tg['proj_W'], _row(tg['proj_b']), blk)
    table = jnp.concatenate([fproj, time_pts, zpad_tgt], axis=-1)
    gath = _sc_gather(table.reshape(bsz * n, _D), idx_tgt, _D, nc_tgt, _GRP)
    gath = gath[:bsz * nq * _K].reshape(bsz, nq, _K, _D)
    wt = {
        'ws': tg['w_Ws'][0][:16], 'wc': tg['w_Ws'][0][16:],
        'wb1': _row(tg['w_bs'][0]), 'ww2': tg['w_Ws'][1],
        'wb2': _row(tg['w_bs'][1]), 'fw1': tg['f_Ws'][0],
        'fb1': _row(tg['f_bs'][0]), 'fw2': tg['f_Ws'][1],
        'fb2': _row(tg['f_bs'][1]),
    }
    return _target_conv(gath, query_pts, wt, 512)


# speedup vs baseline: 4.2540x; 1.5520x over previous
import functools

import numpy as np
import jax
import jax.numpy as jnp
from jax import lax
from jax.experimental import pallas as pl
from jax.experimental.pallas import tpu as pltpu
from jax.experimental.pallas import tpu_sc as plsc

_EIG = 16
_K = 16
_NW = 32
_CH = 128
_GRP = 4
_D = 128

_F2PI = (np.float64(np.float32(2.0 * np.pi)) *
         (2.0 ** np.arange(16))).astype(np.float32)


def _full_spec(shape):
    nd = len(shape)
    return pl.BlockSpec(shape, lambda *_: (0,) * nd)


def _sc_gather(table, idx, d, n_chunks, group):
    m = idx.shape[0]
    per_w = n_chunks * _CH
    assert m == _NW * per_w, (m, n_chunks)
    assert n_chunks % group == 0
    n_groups = n_chunks // group
    mesh = plsc.VectorSubcoreMesh(core_axis_name="c", subcore_axis_name="s")

    @functools.partial(
        pl.kernel,
        mesh=mesh,
        out_type=jax.ShapeDtypeStruct((m, d), jnp.float32),
        scratch_types=[
            pltpu.VMEM((group, _CH), jnp.int32),
            pltpu.VMEM((group, _CH, d), jnp.float32),
            pltpu.SemaphoreType.DMA,
        ],
    )
    def gather_kernel(table_hbm, idx_hbm, out_hbm, idx_v, rows_v, sem):
        wid = lax.axis_index("s") * 2 + lax.axis_index("c")
        base = wid * per_w

        def body(g, carry):
            goff = base + g * (group * _CH)
            handles = []
            for b in range(group):
                pltpu.sync_copy(idx_hbm.at[pl.ds(goff + b * _CH, _CH)],
                                idx_v.at[b])
                handles.append(
                    pltpu.async_copy(table_hbm.at[idx_v.at[b]],
                                     rows_v.at[b], sem))
            for h in handles:
                h.wait()
            for b in range(group):
                pltpu.sync_copy(rows_v.at[b],
                                out_hbm.at[pl.ds(goff + b * _CH, _CH)])
            return carry

        lax.fori_loop(0, n_groups, body, 0)

    return gather_kernel(table, idx)


def _spectral_coeffs(x, e, theta2):
    bsz, n, c = x.shape

    def body(x_ref, e_ref, th_ref, out_ref):
        co = lax.dot_general(e_ref[0], x_ref[0], (((0,), (0,)), ((), ())),
                             preferred_element_type=jnp.float32)
        out_ref[0] = co * th_ref[...]

    return pl.pallas_call(
        body,
        grid=(bsz,),
        in_specs=[
            pl.BlockSpec((1, n, c), lambda i: (i, 0, 0)),
            pl.BlockSpec((1, n, _EIG), lambda i: (i, 0, 0)),
            _full_spec((_EIG, 1)),
        ],
        out_specs=pl.BlockSpec((1, _EIG, c), lambda i: (i, 0, 0)),
        out_shape=jax.ShapeDtypeStruct((bsz, _EIG, c), jnp.float32),
    )(x, e, theta2)


def _layer_pre(x, filt, e, sp, w, blk):
    bsz, n, c = x.shape

    def body(x_ref, f_ref, e_ref, sp_ref, w1x, w1p, w1s, b1, w2, b2,
             px, pn, pb, nei_ref, fp_ref):
        xs = x_ref[0]
        pt = jnp.dot(e_ref[0], f_ref[0], preferred_element_type=jnp.float32)
        h = (jnp.dot(xs, w1x[...], preferred_element_type=jnp.float32)
             + jnp.dot(pt, w1p[...], preferred_element_type=jnp.float32)
             + jnp.dot(sp_ref[0], w1s[...], preferred_element_type=jnp.float32)
             + b1[...])
        h = jnp.maximum(h, 0.0)
        nei = jnp.dot(h, w2[...], preferred_element_type=jnp.float32) + b2[...]
        nei_ref[0] = nei
        fp_ref[0] = (jnp.dot(xs, px[...], preferred_element_type=jnp.float32)
                     + jnp.dot(nei, pn[...], preferred_element_type=jnp.float32)
                     + pb[...])

    nb = n // blk
    return pl.pallas_call(
        body,
        grid=(bsz, nb),
        in_specs=[
            pl.BlockSpec((1, blk, c), lambda i, j: (i, j, 0)),
            pl.BlockSpec((1, _EIG, c), lambda i, j: (i, 0, 0)),
            pl.BlockSpec((1, blk, _EIG), lambda i, j: (i, j, 0)),
            pl.BlockSpec((1, blk, 3), lambda i, j: (i, j, 0)),
            _full_spec(w['w1x'].shape), _full_spec(w['w1p'].shape),
            _full_spec(w['w1s'].shape), _full_spec(w['b1'].shape),
            _full_spec(w['w2'].shape), _full_spec(w['b2'].shape),
            _full_spec(w['px'].shape), _full_spec(w['pn'].shape),
            _full_spec(w['pb'].shape),
        ],
        out_specs=[
            pl.BlockSpec((1, blk, 64), lambda i, j: (i, j, 0)),
            pl.BlockSpec((1, blk, _EIG), lambda i, j: (i, j, 0)),
        ],
        out_shape=[
            jax.ShapeDtypeStruct((bsz, n, 64), jnp.float32),
            jax.ShapeDtypeStruct((bsz, n, _EIG), jnp.float32),
        ],
    )(x, filt, e, sp, w['w1x'], w['w1p'], w['w1s'], w['b1'], w['w2'], w['b2'],
      w['px'], w['pn'], w['pb'])


def _layer_post(x, nei, gath, t, w, blk):
    bsz, n, c = x.shape

    def body(x_ref, nei_ref, g_ref, t_ref, fq_ref, ws, wc, wb1, ww2, wb2,
             fw1, fb1, fw2, fb2, c1x, c1n, c1t, cb1, c2w, cb2, out_ref):
        g = g_ref[0]
        fn = g[:, :, :_EIG]
        tn = g[:, :, _EIG:_EIG + 1]
        dt = tn - t_ref[0][:, :, None]
        ang = dt * fq_ref[...][None]
        s = jnp.sin(ang).reshape(blk * _K, 16)
        co = jnp.cos(ang).reshape(blk * _K, 16)
        h = (jnp.dot(s, ws[...], preferred_element_type=jnp.float32)
             + jnp.dot(co, wc[...], preferred_element_type=jnp.float32)
             + wb1[...])
        h = jnp.maximum(h, 0.0)
        wgt = (jnp.dot(h, ww2[...], preferred_element_type=jnp.float32)
               + wb2[...]).reshape(blk, _K, _EIG)
        agg = jnp.sum(wgt * fn, axis=1)
        h2 = jnp.maximum(
            jnp.dot(agg, fw1[...], preferred_element_type=jnp.float32)
            + fb1[...], 0.0)
        tnei = (jnp.dot(h2, fw2[...], preferred_element_type=jnp.float32)
                + fb2[...])
        hh = (jnp.dot(x_ref[0], c1x[...], preferred_element_type=jnp.float32)
              + jnp.dot(nei_ref[0], c1n[...], preferred_element_type=jnp.float32)
              + jnp.dot(tnei, c1t[...], preferred_element_type=jnp.float32)
              + cb1[...])
        hh = jnp.maximum(hh, 0.0)
        out_ref[0] = (jnp.dot(hh, c2w[...], preferred_element_type=jnp.float32)
                      + cb2[...])

    nb = n // blk
    return pl.pallas_call(
        body,
        grid=(bsz, nb),
        in_specs=[
            pl.BlockSpec((1, blk, c), lambda i, j: (i, j, 0)),
            pl.BlockSpec((1, blk, 64), lambda i, j: (i, j, 0)),
            pl.BlockSpec((1, blk, _K, _D), lambda i, j: (i, j, 0, 0)),
            pl.BlockSpec((1, blk, 1), lambda i, j: (i, j, 0)),
            _full_spec((1, 16)),
            _full_spec(w['ws'].shape), _full_spec(w['wc'].shape),
            _full_spec(w['wb1'].shape), _full_spec(w['ww2'].shape),
            _full_spec(w['wb2'].shape), _full_spec(w['fw1'].shape),
            _full_spec(w['fb1'].shape), _full_spec(w['fw2'].shape),
            _full_spec(w['fb2'].shape), _full_spec(w['c1x'].shape),
            _full_spec(w['c1n'].shape), _full_spec(w['c1t'].shape),
            _full_spec(w['cb1'].shape), _full_spec(w['c2'].shape),
            _full_spec(w['cb2'].shape),
        ],
        out_specs=pl.BlockSpec((1, blk, 128), lambda i, j: (i, j, 0)),
        out_shape=jax.ShapeDtypeStruct((bsz, n, 128), jnp.float32),
    )(x, nei, gath, t, jnp.asarray(_F2PI).reshape(1, 16),
      w['ws'], w['wc'], w['wb1'], w['ww2'], w['wb2'],
      w['fw1'], w['fb1'], w['fw2'], w['fb2'], w['c1x'], w['c1n'], w['c1t'],
      w['cb1'], w['c2'], w['cb2'])


def _target_proj(x, pw, pb, blk):
    bsz, n, c = x.shape
    d = pw.shape[1]

    def body(x_ref, pw_ref, pb_ref, out_ref):
        out_ref[0] = (jnp.dot(x_ref[0], pw_ref[...],
                              preferred_element_type=jnp.float32)
                      + pb_ref[...])

    return pl.pallas_call(
        body,
        grid=(bsz, n // blk),
        in_specs=[
            pl.BlockSpec((1, blk, c), lambda i, j: (i, j, 0)),
            _full_spec(pw.shape), _full_spec(pb.shape),
        ],
        out_specs=pl.BlockSpec((1, blk, d), lambda i, j: (i, j, 0)),
        out_shape=jax.ShapeDtypeStruct((bsz, n, d), jnp.float32),
    )(x, pw, pb)


def _target_conv(gath, qt, w, blk):
    bsz, nq = qt.shape[0], qt.shape[1]

    def body(g_ref, t_ref, fq_ref, ws, wc, wb1, ww2, wb2, fw1, fb1, fw2, fb2,
             out_ref):
        g = g_ref[0]
        fn = g[:, :, :32]
        tn = g[:, :, 32:33]
        dt = tn - t_ref[0][:, :, None]
        ang = dt * fq_ref[...][None]
        s = jnp.sin(ang).reshape(blk * _K, 16)
        co = jnp.cos(ang).reshape(blk * _K, 16)
        h = (jnp.dot(s, ws[...], preferred_element_type=jnp.float32)
             + jnp.dot(co, wc[...], preferred_element_type=jnp.float32)
             + wb1[...])
        h = jnp.maximum(h, 0.0)
        wgt = (jnp.dot(h, ww2[...], preferred_element_type=jnp.float32)
               + wb2[...]).reshape(blk, _K, 32)
        agg = jnp.sum(wgt * fn, axis=1)
        h2 = jnp.maximum(
            jnp.dot(agg, fw1[...], preferred_element_type=jnp.float32)
            + fb1[...], 0.0)
        out_ref[0] = (jnp.dot(h2, fw2[...], preferred_element_type=jnp.float32)
                      + fb2[...])

    return pl.pallas_call(
        body,
        grid=(bsz, nq // blk),
        in_specs=[
            pl.BlockSpec((1, blk, _K, _D), lambda i, j: (i, j, 0, 0)),
            pl.BlockSpec((1, blk, 1), lambda i, j: (i, j, 0)),
            _full_spec((1, 16)),
            _full_spec(w['ws'].shape), _full_spec(w['wc'].shape),
            _full_spec(w['wb1'].shape), _full_spec(w['ww2'].shape),
            _full_spec(w['wb2'].shape), _full_spec(w['fw1'].shape),
            _full_spec(w['fb1'].shape), _full_spec(w['fw2'].shape),
            _full_spec(w['fb2'].shape),
        ],
        out_specs=pl.BlockSpec((1, blk, 128), lambda i, j: (i, j, 0)),
        out_shape=jax.ShapeDtypeStruct((bsz, nq, 128), jnp.float32),
    )(gath, qt, jnp.asarray(_F2PI).reshape(1, 16),
      w['ws'], w['wc'], w['wb1'], w['ww2'], w['wb2'],
      w['fw1'], w['fb1'], w['fw2'], w['fb2'])


def _row(b):
    return b.reshape(1, -1)


def _prep_layer(lp, c):
    s, t, cb = lp['spec'], lp['time'], lp['comb']
    w = {}
    w['w1x'] = s['Ws'][0][:c]
    w['w1p'] = s['Ws'][0][c:2 * c]
    w['w1s'] = s['Ws'][0][2 * c:]
    w['b1'] = _row(s['bs'][0])
    w['w2'] = s['Ws'][1]
    w['b2'] = _row(s['bs'][1])
    w['px'] = t['proj_W'][:c]
    w['pn'] = t['proj_W'][c:]
    w['pb'] = _row(t['proj_b'])
    w['ws'] = t['w_Ws'][0][:16]
    w['wc'] = t['w_Ws'][0][16:]
    w['wb1'] = _row(t['w_bs'][0])
    w['ww2'] = t['w_Ws'][1]
    w['wb2'] = _row(t['w_bs'][1])
    w['fw1'] = t['f_Ws'][0]
    w['fb1'] = _row(t['f_bs'][0])
    w['fw2'] = t['f_Ws'][1]
    w['fb2'] = _row(t['f_bs'][1])
    w['c1x'] = cb['Ws'][0][:c]
    w['c1n'] = cb['Ws'][0][c:c + 64]
    w['c1t'] = cb['Ws'][0][c + 64:]
    w['cb1'] = _row(cb['bs'][0])
    w['c2'] = cb['Ws'][1]
    w['cb2'] = _row(cb['bs'][1])
    return w


def _pad_idx(idx, n_chunks):
    m = _NW * n_chunks * _CH
    return jnp.concatenate(
        [idx, jnp.zeros((m - idx.shape[0],), jnp.int32)])


def kernel(data, ids, space_pts, time_pts, query_pts, eig, params):
    bsz, n, _ = data.shape
    nq = query_pts.shape[1]

    k_t = time_pts[..., 0]
    order = jnp.argsort(k_t, axis=1)
    sorted_t = jnp.take_along_axis(k_t, order, axis=1)

    def window_idx(q_t):
        pos = jax.vmap(lambda s, q: jnp.searchsorted(s, q))(sorted_t, q_t)
        start = jnp.clip(pos - _K // 2, 0, n - _K)
        return start[:, :, None] + jnp.arange(_K)[None, None, :]

    boff = (jnp.arange(bsz, dtype=jnp.int32) * n)[:, None, None]
    idx_self = (window_idx(k_t).astype(jnp.int32) + boff).reshape(-1)
    idx_tgt = (window_idx(query_pts[..., 0]).astype(jnp.int32)
               + boff).reshape(-1)
    idx_ord = (order.astype(jnp.int32) + boff[..., 0]).reshape(-1)
    nc_self = (idx_self.shape[0] + _NW * _CH - 1) // (_NW * _CH)
    nc_self += (-nc_self) % _GRP
    nc_tgt = (idx_tgt.shape[0] + _NW * _CH - 1) // (_NW * _CH)
    nc_tgt += (-nc_tgt) % _GRP
    idx_self = _pad_idx(idx_self, nc_self)
    idx_tgt = _pad_idx(idx_tgt, nc_tgt)

    nc_e = (bsz * n + _NW * _CH - 1) // (_NW * _CH)
    nc_e += (-nc_e) % _GRP
    idx_e = _pad_idx(ids.reshape(-1), nc_e)
    idx_ord = _pad_idx(idx_ord, nc_e)

    eig_pad = jnp.concatenate(
        [eig, jnp.zeros((eig.shape[0], _D - _EIG), jnp.float32)], axis=-1)
    e = _sc_gather(eig_pad, idx_e, _D, nc_e, _GRP)[:bsz * n, :_EIG]
    e = e.reshape(bsz, n, _EIG)

    blk = 1000
    blk_post = 400
    zpad_self = jnp.zeros((bsz, n, _D - _EIG - 1), jnp.float32)
    zpad_tgt = jnp.zeros((bsz, n, _D - 33), jnp.float32)
    x = data
    for lp in params['layers']:
        c = x.shape[2]
        w = _prep_layer(lp, c)
        filt = _spectral_coeffs(x, e, lp['spec']['theta'].reshape(_EIG, 1))
        nei, fproj = _layer_pre(x, filt, e, space_pts, w, blk)
        table = jnp.concatenate([fproj, time_pts, zpad_self], axis=-1)
        tsort = _sc_gather(table.reshape(bsz * n, _D), idx_ord, _D,
                           nc_e, _GRP)
        gath = _sc_gather(tsort, idx_self, _D, nc_self, _GRP)
        gath = gath[:bsz * n * _K].reshape(bsz, n, _K, _D)
        x = _layer_post(x, nei, gath, time_pts, w, blk_post)

    tg = params['target']
    fproj = _target_proj(x, tg['proj_W'], _row(tg['proj_b']), blk)
    table = jnp.concatenate([fproj, time_pts, zpad_tgt], axis=-1)
    tsort = _sc_gather(table.reshape(bsz * n, _D), idx_ord, _D, nc_e, _GRP)
    gath = _sc_gather(tsort, idx_tgt, _D, nc_tgt, _GRP)
    gath = gath[:bsz * nq * _K].reshape(bsz, nq, _K, _D)
    wt = {
        'ws': tg['w_Ws'][0][:16], 'wc': tg['w_Ws'][0][16:],
        'wb1': _row(tg['w_bs'][0]), 'ww2': tg['w_Ws'][1],
        'wb2': _row(tg['w_bs'][1]), 'fw1': tg['f_Ws'][0],
        'fb1': _row(tg['f_bs'][0]), 'fw2': tg['f_Ws'][1],
        'fb2': _row(tg['f_bs'][1]),
    }
    return _target_conv(gath, query_pts, wt, 512)


# speedup vs baseline: 5.4591x; 1.2833x over previous
import functools

import numpy as np
import jax
import jax.numpy as jnp
from jax import lax
from jax.experimental import pallas as pl
from jax.experimental.pallas import tpu as pltpu
from jax.experimental.pallas import tpu_sc as plsc

_EIG = 16
_K = 16
_NW = 32
_CH = 128
_GRP = 4
_D = 128

_F2PI = (np.float64(np.float32(2.0 * np.pi)) *
         (2.0 ** np.arange(16))).astype(np.float32)


def _full_spec(shape):
    nd = len(shape)
    return pl.BlockSpec(shape, lambda *_: (0,) * nd)


def _sc_gather(table, idx, d, n_chunks, group):
    m = idx.shape[0]
    per_w = n_chunks * _CH
    assert m == _NW * per_w, (m, n_chunks)
    assert n_chunks % group == 0
    n_groups = n_chunks // group
    mesh = plsc.VectorSubcoreMesh(core_axis_name="c", subcore_axis_name="s")

    @functools.partial(
        pl.kernel,
        mesh=mesh,
        out_type=jax.ShapeDtypeStruct((m, d), jnp.float32),
        scratch_types=[
            pltpu.VMEM((group, _CH), jnp.int32),
            pltpu.VMEM((group, _CH, d), jnp.float32),
            pltpu.SemaphoreType.DMA,
        ],
    )
    def gather_kernel(table_hbm, idx_hbm, out_hbm, idx_v, rows_v, sem):
        wid = lax.axis_index("s") * 2 + lax.axis_index("c")
        base = wid * per_w

        def body(g, carry):
            goff = base + g * (group * _CH)
            handles = []
            for b in range(group):
                pltpu.sync_copy(idx_hbm.at[pl.ds(goff + b * _CH, _CH)],
                                idx_v.at[b])
                handles.append(
                    pltpu.async_copy(table_hbm.at[idx_v.at[b]],
                                     rows_v.at[b], sem))
            for h in handles:
                h.wait()
            for b in range(group):
                pltpu.sync_copy(rows_v.at[b],
                                out_hbm.at[pl.ds(goff + b * _CH, _CH)])
            return carry

        lax.fori_loop(0, n_groups, body, 0)

    return gather_kernel(table, idx)


def _spectral_coeffs(x, e, theta2):
    bsz, n, c = x.shape

    def body(x_ref, e_ref, th_ref, out_ref):
        co = lax.dot_general(e_ref[0], x_ref[0], (((0,), (0,)), ((), ())),
                             preferred_element_type=jnp.float32)
        out_ref[0] = co * th_ref[...]

    return pl.pallas_call(
        body,
        grid=(bsz,),
        in_specs=[
            pl.BlockSpec((1, n, c), lambda i: (i, 0, 0)),
            pl.BlockSpec((1, n, _EIG), lambda i: (i, 0, 0)),
            _full_spec((_EIG, 1)),
        ],
        out_specs=pl.BlockSpec((1, _EIG, c), lambda i: (i, 0, 0)),
        out_shape=jax.ShapeDtypeStruct((bsz, _EIG, c), jnp.float32),
    )(x, e, theta2)


def _layer_pre(x, filt, e, sp, w, blk):
    bsz, n, c = x.shape

    def body(x_ref, f_ref, e_ref, sp_ref, w1x, w1p, w1s, b1, w2, b2,
             px, pn, pb, nei_ref, fp_ref):
        xs = x_ref[0]
        pt = jnp.dot(e_ref[0], f_ref[0], preferred_element_type=jnp.float32)
        h = (jnp.dot(xs, w1x[...], preferred_element_type=jnp.float32)
             + jnp.dot(pt, w1p[...], preferred_element_type=jnp.float32)
             + jnp.dot(sp_ref[0], w1s[...], preferred_element_type=jnp.float32)
             + b1[...])
        h = jnp.maximum(h, 0.0)
        nei = jnp.dot(h, w2[...], preferred_element_type=jnp.float32) + b2[...]
        nei_ref[0] = nei
        fp_ref[0] = (jnp.dot(xs, px[...], preferred_element_type=jnp.float32)
                     + jnp.dot(nei, pn[...], preferred_element_type=jnp.float32)
                     + pb[...])

    nb = n // blk
    return pl.pallas_call(
        body,
        grid=(bsz, nb),
        in_specs=[
            pl.BlockSpec((1, blk, c), lambda i, j: (i, j, 0)),
            pl.BlockSpec((1, _EIG, c), lambda i, j: (i, 0, 0)),
            pl.BlockSpec((1, blk, _EIG), lambda i, j: (i, j, 0)),
            pl.BlockSpec((1, blk, 3), lambda i, j: (i, j, 0)),
            _full_spec(w['w1x'].shape), _full_spec(w['w1p'].shape),
            _full_spec(w['w1s'].shape), _full_spec(w['b1'].shape),
            _full_spec(w['w2'].shape), _full_spec(w['b2'].shape),
            _full_spec(w['px'].shape), _full_spec(w['pn'].shape),
            _full_spec(w['pb'].shape),
        ],
        out_specs=[
            pl.BlockSpec((1, blk, 64), lambda i, j: (i, j, 0)),
            pl.BlockSpec((1, blk, _EIG), lambda i, j: (i, j, 0)),
        ],
        out_shape=[
            jax.ShapeDtypeStruct((bsz, n, 64), jnp.float32),
            jax.ShapeDtypeStruct((bsz, n, _EIG), jnp.float32),
        ],
    )(x, filt, e, sp, w['w1x'], w['w1p'], w['w1s'], w['b1'], w['w2'], w['b2'],
      w['px'], w['pn'], w['pb'])


def _layer_post(x, nei, gath, t, w, blk):
    bsz, n, c = x.shape

    def body(x_ref, nei_ref, g_ref, t_ref, fq_ref, ws, wc, wb1, ww2, wb2,
             fw1, fb1, fw2, fb2, c1x, c1n, c1t, cb1, c2w, cb2, out_ref):
        g = g_ref[0]
        fn = g[:, :, :_EIG]
        tn = g[:, :, _EIG:_EIG + 1]
        dt = tn - t_ref[0][:, :, None]
        ang = dt * fq_ref[...][None]
        s = jnp.sin(ang).reshape(blk * _K, 16)
        co = jnp.cos(ang).reshape(blk * _K, 16)
        h = (jnp.dot(s, ws[...], preferred_element_type=jnp.float32)
             + jnp.dot(co, wc[...], preferred_element_type=jnp.float32)
             + wb1[...])
        h = jnp.maximum(h, 0.0)
        wgt = (jnp.dot(h, ww2[...], preferred_element_type=jnp.float32)
               + wb2[...]).reshape(blk, _K, _EIG)
        agg = jnp.sum(wgt * fn, axis=1)
        h2 = jnp.maximum(
            jnp.dot(agg, fw1[...], preferred_element_type=jnp.float32)
            + fb1[...], 0.0)
        tnei = (jnp.dot(h2, fw2[...], preferred_element_type=jnp.float32)
                + fb2[...])
        hh = (jnp.dot(x_ref[0], c1x[...], preferred_element_type=jnp.float32)
              + jnp.dot(nei_ref[0], c1n[...], preferred_element_type=jnp.float32)
              + jnp.dot(tnei, c1t[...], preferred_element_type=jnp.float32)
              + cb1[...])
        hh = jnp.maximum(hh, 0.0)
        out_ref[0] = (jnp.dot(hh, c2w[...], preferred_element_type=jnp.float32)
                      + cb2[...])

    nb = n // blk
    return pl.pallas_call(
        body,
        grid=(bsz, nb),
        in_specs=[
            pl.BlockSpec((1, blk, c), lambda i, j: (i, j, 0)),
            pl.BlockSpec((1, blk, 64), lambda i, j: (i, j, 0)),
            pl.BlockSpec((1, blk, _K, _D), lambda i, j: (i, j, 0, 0)),
            pl.BlockSpec((1, blk, 1), lambda i, j: (i, j, 0)),
            _full_spec((1, 16)),
            _full_spec(w['ws'].shape), _full_spec(w['wc'].shape),
            _full_spec(w['wb1'].shape), _full_spec(w['ww2'].shape),
            _full_spec(w['wb2'].shape), _full_spec(w['fw1'].shape),
            _full_spec(w['fb1'].shape), _full_spec(w['fw2'].shape),
            _full_spec(w['fb2'].shape), _full_spec(w['c1x'].shape),
            _full_spec(w['c1n'].shape), _full_spec(w['c1t'].shape),
            _full_spec(w['cb1'].shape), _full_spec(w['c2'].shape),
            _full_spec(w['cb2'].shape),
        ],
        out_specs=pl.BlockSpec((1, blk, 128), lambda i, j: (i, j, 0)),
        out_shape=jax.ShapeDtypeStruct((bsz, n, 128), jnp.float32),
    )(x, nei, gath, t, jnp.asarray(_F2PI).reshape(1, 16),
      w['ws'], w['wc'], w['wb1'], w['ww2'], w['wb2'],
      w['fw1'], w['fb1'], w['fw2'], w['fb2'], w['c1x'], w['c1n'], w['c1t'],
      w['cb1'], w['c2'], w['cb2'])


def _target_proj(x, pw, pb, blk):
    bsz, n, c = x.shape
    d = pw.shape[1]

    def body(x_ref, pw_ref, pb_ref, out_ref):
        out_ref[0] = (jnp.dot(x_ref[0], pw_ref[...],
                              preferred_element_type=jnp.float32)
                      + pb_ref[...])

    return pl.pallas_call(
        body,
        grid=(bsz, n // blk),
        in_specs=[
            pl.BlockSpec((1, blk, c), lambda i, j: (i, j, 0)),
            _full_spec(pw.shape), _full_spec(pb.shape),
        ],
        out_specs=pl.BlockSpec((1, blk, d), lambda i, j: (i, j, 0)),
        out_shape=jax.ShapeDtypeStruct((bsz, n, d), jnp.float32),
    )(x, pw, pb)


def _target_conv(gath, qt, w, blk):
    bsz, nq = qt.shape[0], qt.shape[1]

    def body(g_ref, t_ref, fq_ref, ws, wc, wb1, ww2, wb2, fw1, fb1, fw2, fb2,
             out_ref):
        g = g_ref[0]
        fn = g[:, :, :32]
        tn = g[:, :, 32:33]
        dt = tn - t_ref[0][:, :, None]
        ang = dt * fq_ref[...][None]
        s = jnp.sin(ang).reshape(blk * _K, 16)
        co = jnp.cos(ang).reshape(blk * _K, 16)
        h = (jnp.dot(s, ws[...], preferred_element_type=jnp.float32)
             + jnp.dot(co, wc[...], preferred_element_type=jnp.float32)
             + wb1[...])
        h = jnp.maximum(h, 0.0)
        wgt = (jnp.dot(h, ww2[...], preferred_element_type=jnp.float32)
               + wb2[...]).reshape(blk, _K, 32)
        agg = jnp.sum(wgt * fn, axis=1)
        h2 = jnp.maximum(
            jnp.dot(agg, fw1[...], preferred_element_type=jnp.float32)
            + fb1[...], 0.0)
        out_ref[0] = (jnp.dot(h2, fw2[...], preferred_element_type=jnp.float32)
                      + fb2[...])

    return pl.pallas_call(
        body,
        grid=(bsz, nq // blk),
        in_specs=[
            pl.BlockSpec((1, blk, _K, _D), lambda i, j: (i, j, 0, 0)),
            pl.BlockSpec((1, blk, 1), lambda i, j: (i, j, 0)),
            _full_spec((1, 16)),
            _full_spec(w['ws'].shape), _full_spec(w['wc'].shape),
            _full_spec(w['wb1'].shape), _full_spec(w['ww2'].shape),
            _full_spec(w['wb2'].shape), _full_spec(w['fw1'].shape),
            _full_spec(w['fb1'].shape), _full_spec(w['fw2'].shape),
            _full_spec(w['fb2'].shape),
        ],
        out_specs=pl.BlockSpec((1, blk, 128), lambda i, j: (i, j, 0)),
        out_shape=jax.ShapeDtypeStruct((bsz, nq, 128), jnp.float32),
    )(gath, qt, jnp.asarray(_F2PI).reshape(1, 16),
      w['ws'], w['wc'], w['wb1'], w['ww2'], w['wb2'],
      w['fw1'], w['fb1'], w['fw2'], w['fb2'])


def _row(b):
    return b.reshape(1, -1)


def _prep_layer(lp, c):
    s, t, cb = lp['spec'], lp['time'], lp['comb']
    w = {}
    w['w1x'] = s['Ws'][0][:c]
    w['w1p'] = s['Ws'][0][c:2 * c]
    w['w1s'] = s['Ws'][0][2 * c:]
    w['b1'] = _row(s['bs'][0])
    w['w2'] = s['Ws'][1]
    w['b2'] = _row(s['bs'][1])
    w['px'] = t['proj_W'][:c]
    w['pn'] = t['proj_W'][c:]
    w['pb'] = _row(t['proj_b'])
    w['ws'] = t['w_Ws'][0][:16]
    w['wc'] = t['w_Ws'][0][16:]
    w['wb1'] = _row(t['w_bs'][0])
    w['ww2'] = t['w_Ws'][1]
    w['wb2'] = _row(t['w_bs'][1])
    w['fw1'] = t['f_Ws'][0]
    w['fb1'] = _row(t['f_bs'][0])
    w['fw2'] = t['f_Ws'][1]
    w['fb2'] = _row(t['f_bs'][1])
    w['c1x'] = cb['Ws'][0][:c]
    w['c1n'] = cb['Ws'][0][c:c + 64]
    w['c1t'] = cb['Ws'][0][c + 64:]
    w['cb1'] = _row(cb['bs'][0])
    w['c2'] = cb['Ws'][1]
    w['cb2'] = _row(cb['bs'][1])
    return w


def _pad_idx(idx, n_chunks):
    m = _NW * n_chunks * _CH
    return jnp.concatenate(
        [idx, jnp.zeros((m - idx.shape[0],), jnp.int32)])


def kernel(data, ids, space_pts, time_pts, query_pts, eig, params):
    bsz, n, _ = data.shape
    nq = query_pts.shape[1]

    k_t = time_pts[..., 0]
    order = jnp.argsort(k_t, axis=1)
    sorted_t = jnp.take_along_axis(k_t, order, axis=1)

    def window_idx(q_t):
        pos = jax.vmap(lambda s, q: jnp.searchsorted(s, q))(sorted_t, q_t)
        start = jnp.clip(pos - _K // 2, 0, n - _K)
        return start[:, :, None] + jnp.arange(_K)[None, None, :]

    boff = (jnp.arange(bsz, dtype=jnp.int32) * n)[:, None, None]
    idx_self = (window_idx(sorted_t).astype(jnp.int32) + boff).reshape(-1)
    idx_tgt = (window_idx(query_pts[..., 0]).astype(jnp.int32)
               + boff).reshape(-1)
    nc_self = (idx_self.shape[0] + _NW * _CH - 1) // (_NW * _CH)
    nc_self += (-nc_self) % _GRP
    nc_tgt = (idx_tgt.shape[0] + _NW * _CH - 1) // (_NW * _CH)
    nc_tgt += (-nc_tgt) % _GRP
    idx_self = _pad_idx(idx_self, nc_self)
    idx_tgt = _pad_idx(idx_tgt, nc_tgt)

    x = jnp.take_along_axis(data, order[..., None], axis=1)
    sp_s = jnp.take_along_axis(space_pts, order[..., None], axis=1)
    ids_s = jnp.take_along_axis(ids, order, axis=1)
    st_col = sorted_t[..., None]

    nc_e = (bsz * n + _NW * _CH - 1) // (_NW * _CH)
    nc_e += (-nc_e) % _GRP
    idx_e = _pad_idx(ids_s.reshape(-1), nc_e)

    eig_pad = jnp.concatenate(
        [eig, jnp.zeros((eig.shape[0], _D - _EIG), jnp.float32)], axis=-1)
    e = _sc_gather(eig_pad, idx_e, _D, nc_e, _GRP)[:bsz * n, :_EIG]
    e = e.reshape(bsz, n, _EIG)

    blk = 1000
    blk_post = 400
    zpad_self = jnp.zeros((bsz, n, _D - _EIG - 1), jnp.float32)
    zpad_tgt = jnp.zeros((bsz, n, _D - 33), jnp.float32)
    for lp in params['layers']:
        c = x.shape[2]
        w = _prep_layer(lp, c)
        filt = _spectral_coeffs(x, e, lp['spec']['theta'].reshape(_EIG, 1))
        nei, fproj = _layer_pre(x, filt, e, sp_s, w, blk)
        table = jnp.concatenate([fproj, st_col, zpad_self], axis=-1)
        gath = _sc_gather(table.reshape(bsz * n, _D), idx_self, _D,
                          nc_self, _GRP)
        gath = gath[:bsz * n * _K].reshape(bsz, n, _K, _D)
        x = _layer_post(x, nei, gath, st_col, w, blk_post)

    tg = params['target']
    fproj = _target_proj(x, tg['proj_W'], _row(tg['proj_b']), blk)
    table = jnp.concatenate([fproj, st_col, zpad_tgt], axis=-1)
    gath = _sc_gather(table.reshape(bsz * n, _D), idx_tgt, _D, nc_tgt, _GRP)
    gath = gath[:bsz * nq * _K].reshape(bsz, nq, _K, _D)
    wt = {
        'ws': tg['w_Ws'][0][:16], 'wc': tg['w_Ws'][0][16:],
        'wb1': _row(tg['w_bs'][0]), 'ww2': tg['w_Ws'][1],
        'wb2': _row(tg['w_bs'][1]), 'fw1': tg['f_Ws'][0],
        'fb1': _row(tg['f_bs'][0]), 'fw2': tg['f_Ws'][1],
        'fb2': _row(tg['f_bs'][1]),
    }
    return _target_conv(gath, query_pts, wt, 512)
